# baseline (speedup 1.0000x reference)
import functools

import jax
import jax.numpy as jnp
from jax.experimental import pallas as pl
from jax.experimental.pallas import tpu as pltpu

_HEADS = 7
_D = 64


def _pad_rows(a, bm):
    m = a.shape[0]
    mp = ((m + bm - 1) // bm) * bm
    if mp == m:
        return a
    return jnp.pad(a, ((0, mp - m), (0, 0)))


def _mlp3_block(x_ref, w1_ref, b1_ref, w2_ref, b2_ref, w3_ref, b3_ref, o_ref):
    h = jnp.maximum(jnp.dot(x_ref[:], w1_ref[:], preferred_element_type=jnp.float32) + b1_ref[:], 0.0)
    h = jnp.maximum(jnp.dot(h, w2_ref[:], preferred_element_type=jnp.float32) + b2_ref[:], 0.0)
    h = jnp.maximum(jnp.dot(h, w3_ref[:], preferred_element_type=jnp.float32) + b3_ref[:], 0.0)
    o_ref[:] = h


def _fc1_forward(x, fc1):
    (w1, b1), (w2, b2), (w3, b3) = fc1
    bm = 512
    xp = _pad_rows(x, bm)
    mp = xp.shape[0]
    grid = (mp // bm,)
    full = lambda i: (0, 0)
    out = pl.pallas_call(
        _mlp3_block,
        grid=grid,
        in_specs=[
            pl.BlockSpec((bm, xp.shape[1]), lambda i: (i, 0)),
            pl.BlockSpec((w1.shape[1], w1.shape[0]), full),
            pl.BlockSpec((1, w1.shape[0]), full),
            pl.BlockSpec((w2.shape[1], w2.shape[0]), full),
            pl.BlockSpec((1, w2.shape[0]), full),
            pl.BlockSpec((w3.shape[1], w3.shape[0]), full),
            pl.BlockSpec((1, w3.shape[0]), full),
        ],
        out_specs=pl.BlockSpec((bm, w3.shape[0]), lambda i: (i, 0)),
        out_shape=jax.ShapeDtypeStruct((mp, w3.shape[0]), jnp.float32),
    )(xp, w1.T, b1[None, :], w2.T, b2[None, :], w3.T, b3[None, :])
    return out[: x.shape[0]]


def _proj_block(h_ref, w_ref, b_ref, o_ref):
    o_ref[:] = jnp.dot(h_ref[:], w_ref[:], preferred_element_type=jnp.float32) + b_ref[:]


def _conv_proj(h, wl, bl, wr, br):
    wcat = jnp.concatenate([wl, wr], axis=0)
    bcat = jnp.concatenate([bl, br], axis=0)
    bm = 1024
    hp = _pad_rows(h, bm)
    mp = hp.shape[0]
    out = pl.pallas_call(
        _proj_block,
        grid=(mp // bm,),
        in_specs=[
            pl.BlockSpec((bm, h.shape[1]), lambda i: (i, 0)),
            pl.BlockSpec((h.shape[1], wcat.shape[0]), lambda i: (0, 0)),
            pl.BlockSpec((1, wcat.shape[0]), lambda i: (0, 0)),
        ],
        out_specs=pl.BlockSpec((bm, wcat.shape[0]), lambda i: (i, 0)),
        out_shape=jax.ShapeDtypeStruct((mp, wcat.shape[0]), jnp.float32),
    )(hp, wcat.T, bcat[None, :])
    return out[: h.shape[0]]


def _fc2_block(h_ref, w1_ref, b1_ref, w2_ref, b2_ref, w3_ref, b3_ref, o_ref):
    h = jnp.maximum(h_ref[:], 0.0)
    h = jnp.dot(h, w1_ref[:], preferred_element_type=jnp.float32) + b1_ref[:]
    h = jnp.maximum(h, 0.0)
    h = jnp.dot(h, w2_ref[:], preferred_element_type=jnp.float32) + b2_ref[:]
    h = jnp.maximum(h, 0.0)
    h = jnp.dot(h, w3_ref[:], preferred_element_type=jnp.float32) + b3_ref[:]
    m = jnp.max(h, axis=1, keepdims=True)
    lse = jnp.log(jnp.sum(jnp.exp(h - m), axis=1, keepdims=True)) + m
    o_ref[:] = h - lse


def _fc2_forward(h, fc2):
    (w1, b1), (w2, b2), (w3, b3) = fc2
    bm = 512
    hp = _pad_rows(h, bm)
    mp = hp.shape[0]
    full = lambda i: (0, 0)
    out = pl.pallas_call(
        _fc2_block,
        grid=(mp // bm,),
        in_specs=[
            pl.BlockSpec((bm, h.shape[1]), lambda i: (i, 0)),
            pl.BlockSpec((w1.shape[1], w1.shape[0]), full),
            pl.BlockSpec((1, w1.shape[0]), full),
            pl.BlockSpec((w2.shape[1], w2.shape[0]), full),
            pl.BlockSpec((1, w2.shape[0]), full),
            pl.BlockSpec((w3.shape[1], w3.shape[0]), full),
            pl.BlockSpec((1, w3.shape[0]), full),
        ],
        out_specs=pl.BlockSpec((bm, w3.shape[0]), lambda i: (i, 0)),
        out_shape=jax.ShapeDtypeStruct((mp, w3.shape[0]), jnp.float32),
    )(hp, w1.T, b1[None, :], w2.T, b2[None, :], w3.T, b3[None, :])
    return out[: h.shape[0]]


def _edge_phase(xlr, src, dst, att, bias, n):
    xl = xlr[:, : _HEADS * _D].reshape(n, _HEADS, _D)
    xr = xlr[:, _HEADS * _D :].reshape(n, _HEADS, _D)
    x_j = xl[src]
    x_i = xr[dst]
    e = jax.nn.leaky_relu(x_i + x_j, negative_slope=0.2)
    alpha = jnp.einsum("ehd,hd->eh", e, att)
    amax = jax.ops.segment_max(alpha, dst, num_segments=n)
    ex = jnp.exp(alpha - amax[dst])
    denom = jax.ops.segment_sum(ex, dst, num_segments=n)
    alpha = ex / (denom[dst] + 1e-16)
    msg = x_j * alpha[:, :, None]
    out = jax.ops.segment_sum(msg, dst, num_segments=n)
    out = out.mean(axis=1)
    return out + bias


def kernel(x, edge_index, fc1, convs, fc2):
    n = x.shape[0]
    loops = jnp.arange(n, dtype=edge_index.dtype)
    src = jnp.concatenate([edge_index[0], loops])
    dst = jnp.concatenate([edge_index[1], loops])
    h = _fc1_forward(x, fc1)
    for (wl, bl, wr, br, att, bias) in convs:
        xlr = _conv_proj(h, wl, bl, wr, br)
        h = jnp.maximum(_edge_phase(xlr, src, dst, att, bias, n), 0.0)
    return _fc2_forward(h, fc2)


# speedup vs baseline: 20.6557x; 20.6557x over previous
import functools

import jax
import jax.numpy as jnp
from jax import lax
from jax.experimental import pallas as pl
from jax.experimental.pallas import tpu as pltpu
from jax.experimental.pallas import tpu_sc as plsc

_H = 7
_D = 64
_HD = _H * _D
_NW = 32
_NS = 16
_B = 64
_NEG = -1e30
_WPAD = 512


def _pad_rows(a, bm):
    m = a.shape[0]
    mp = ((m + bm - 1) // bm) * bm
    if mp == m:
        return a
    return jnp.pad(a, ((0, mp - m), (0, 0)))


def _mlp3_block(x_ref, w1_ref, b1_ref, w2_ref, b2_ref, w3_ref, b3_ref, o_ref):
    h = jnp.maximum(jnp.dot(x_ref[:], w1_ref[:], preferred_element_type=jnp.float32) + b1_ref[:], 0.0)
    h = jnp.maximum(jnp.dot(h, w2_ref[:], preferred_element_type=jnp.float32) + b2_ref[:], 0.0)
    h = jnp.maximum(jnp.dot(h, w3_ref[:], preferred_element_type=jnp.float32) + b3_ref[:], 0.0)
    o_ref[:] = h


def _fc1_forward(x, fc1):
    (w1, b1), (w2, b2), (w3, b3) = fc1
    bm = 512
    xp = _pad_rows(x, bm)
    mp = xp.shape[0]
    full = lambda i: (0, 0)
    return pl.pallas_call(
        _mlp3_block,
        grid=(mp // bm,),
        in_specs=[
            pl.BlockSpec((bm, xp.shape[1]), lambda i: (i, 0)),
            pl.BlockSpec((w1.shape[1], w1.shape[0]), full),
            pl.BlockSpec((1, w1.shape[0]), full),
            pl.BlockSpec((w2.shape[1], w2.shape[0]), full),
            pl.BlockSpec((1, w2.shape[0]), full),
            pl.BlockSpec((w3.shape[1], w3.shape[0]), full),
            pl.BlockSpec((1, w3.shape[0]), full),
        ],
        out_specs=pl.BlockSpec((bm, w3.shape[0]), lambda i: (i, 0)),
        out_shape=jax.ShapeDtypeStruct((mp, w3.shape[0]), jnp.float32),
    )(xp, w1.T, b1[None, :], w2.T, b2[None, :], w3.T, b3[None, :])


def _proj2_block(h_ref, w_ref, b_ref, ol_ref, or_ref):
    o = jnp.dot(h_ref[:], w_ref[:], preferred_element_type=jnp.float32) + b_ref[:]
    z = jnp.zeros((o.shape[0], _WPAD - _HD), jnp.float32)
    ol_ref[:] = jnp.concatenate([o[:, :_HD], z], axis=1)
    or_ref[:] = jnp.concatenate([o[:, _HD:], z], axis=1)


def _conv_proj(h, wl, bl, wr, br):
    wcat = jnp.concatenate([wl, wr], axis=0)
    bcat = jnp.concatenate([bl, br], axis=0)
    bm = 1024
    hp = _pad_rows(h, bm)
    mp = hp.shape[0]
    return pl.pallas_call(
        _proj2_block,
        grid=(mp // bm,),
        in_specs=[
            pl.BlockSpec((bm, hp.shape[1]), lambda i: (i, 0)),
            pl.BlockSpec((hp.shape[1], 2 * _HD), lambda i: (0, 0)),
            pl.BlockSpec((1, 2 * _HD), lambda i: (0, 0)),
        ],
        out_specs=[
            pl.BlockSpec((bm, _WPAD), lambda i: (i, 0)),
            pl.BlockSpec((bm, _WPAD), lambda i: (i, 0)),
        ],
        out_shape=[
            jax.ShapeDtypeStruct((mp, _WPAD), jnp.float32),
            jax.ShapeDtypeStruct((mp, _WPAD), jnp.float32),
        ],
    )(hp, wcat.T, bcat[None, :])


def _fc2_block(h_ref, w1_ref, b1_ref, w2_ref, b2_ref, w3_ref, b3_ref, o_ref):
    h = jnp.maximum(h_ref[:], 0.0)
    h = jnp.dot(h, w1_ref[:], preferred_element_type=jnp.float32) + b1_ref[:]
    h = jnp.maximum(h, 0.0)
    h = jnp.dot(h, w2_ref[:], preferred_element_type=jnp.float32) + b2_ref[:]
    h = jnp.maximum(h, 0.0)
    h = jnp.dot(h, w3_ref[:], preferred_element_type=jnp.float32) + b3_ref[:]
    m = jnp.max(h, axis=1, keepdims=True)
    lse = jnp.log(jnp.sum(jnp.exp(h - m), axis=1, keepdims=True)) + m
    o_ref[:] = h - lse


def _fc2_forward(h, fc2):
    (w1, b1), (w2, b2), (w3, b3) = fc2
    bm = 512
    hp = _pad_rows(h, bm)
    mp = hp.shape[0]
    full = lambda i: (0, 0)
    out = pl.pallas_call(
        _fc2_block,
        grid=(mp // bm,),
        in_specs=[
            pl.BlockSpec((bm, h.shape[1]), lambda i: (i, 0)),
            pl.BlockSpec((w1.shape[1], w1.shape[0]), full),
            pl.BlockSpec((1, w1.shape[0]), full),
            pl.BlockSpec((w2.shape[1], w2.shape[0]), full),
            pl.BlockSpec((1, w2.shape[0]), full),
            pl.BlockSpec((w3.shape[1], w3.shape[0]), full),
            pl.BlockSpec((1, w3.shape[0]), full),
        ],
        out_specs=pl.BlockSpec((bm, w3.shape[0]), lambda i: (i, 0)),
        out_shape=jax.ShapeDtypeStruct((mp, w3.shape[0]), jnp.float32),
    )(hp, w1.T, b1[None, :], w2.T, b2[None, :], w3.T, b3[None, :])
    return out[: h.shape[0]]


def _masked_lane_sum(lane, h, v):
    return jnp.sum(jnp.where(lane == h, v, 0.0))


def _finalize_row(lane, svec, accs, biasb, ostage, out_ref, dst_idx):
    rcp = 1.0 / svec
    for q in range(4):
        o = biasb[pl.ds(16 * q, 16)]
        for h in range(_H):
            rh = _masked_lane_sum(lane, h, rcp)
            o = o + accs[4 * h + q] * (rh * (1.0 / _H))
        ostage[pl.ds(16 * q, 16)] = jnp.maximum(o, 0.0)
    pltpu.sync_copy(ostage, out_ref.at[pl.ds(dst_idx * _D, _D)])


def _make_edge_kernels(n, eb, nb):
    mesh = plsc.VectorSubcoreMesh(core_axis_name="c", subcore_axis_name="s")

    def edge_body(xl, xr, srcs, dsts, attf, biasf,
                  out_hbm, pidx_hbm, pdata_hbm,
                  idx_s, idx_d, idx_dx, rows_l, rows_r, attb, biasb,
                  ostage, pstage, pmeta, accb, sem_l, sem_r):
        wid = lax.axis_index("c") * _NS + lax.axis_index("s")
        ebase = wid * eb
        pltpu.sync_copy(attf, attb)
        pltpu.sync_copy(biasf, biasb)
        lane = lax.iota(jnp.int32, 16)
        zv = jnp.zeros((16,), jnp.float32)
        negv = jnp.full((16,), _NEG, jnp.float32)
        pmeta[...] = jnp.full((16,), -1, jnp.int32)

        def stash_partial(slot, m, sv):
            pstage[pl.ds(0, 16)] = m
            pstage[pl.ds(16, 16)] = sv
            for j in range(28):
                pstage[pl.ds(32 + 16 * j, 16)] = accb[j]
            pltpu.sync_copy(pstage, pdata_hbm.at[pl.ds((wid * 2 + slot) * 512, 512)])

        def edge_step(e, ec):
            cur, nfl, m, sv = ec
            d = idx_dx[pl.ds(e, 16)][0]
            same = d == cur
            t = zv
            xlv = []
            for h in range(_H):
                p = None
                for q in range(4):
                    j = 4 * h + q
                    vl = rows_l[e, pl.ds(16 * j, 16)]
                    vr = rows_r[e, pl.ds(16 * j, 16)]
                    xlv.append(vl)
                    v = vl + vr
                    v = jnp.maximum(v, 0.0) + 0.2 * jnp.minimum(v, 0.0)
                    av = attb[pl.ds(16 * j, 16)]
                    p = v * av if p is None else p + v * av
                th = jnp.sum(p)
                t = jnp.where(lane == h, th, t)
            do_flush = jnp.logical_not(same) & (cur >= 0)

            @pl.when(do_flush & (nfl == 0))
            def _():
                pmeta[...] = jnp.where(lane == 0, cur, pmeta[...])
                stash_partial(0, m, sv)

            @pl.when(do_flush & (nfl > 0))
            def _():
                _finalize_row(lane, sv, accb, biasb, ostage, out_hbm, cur)

            mprev = jnp.where(same, m, negv)
            sprev = jnp.where(same, sv, zv)
            m2 = jnp.maximum(mprev, t)
            f = jnp.exp(mprev - m2)
            ex = jnp.exp(t - m2)
            s2 = sprev * f + ex
            for h in range(_H):
                fh = _masked_lane_sum(lane, h, f)
                exh = _masked_lane_sum(lane, h, ex)
                for q in range(4):
                    j = 4 * h + q
                    ap = jnp.where(same, accb[j], zv)
                    accb[j] = ap * fh + xlv[j] * exh
            nfl2 = jnp.where(same | (cur < 0), nfl, nfl + 1)
            return (d, nfl2, m2, s2)

        def batch_step(g, carry):
            pltpu.sync_copy(srcs.at[pl.ds(ebase + g * _B, _B)], idx_s)
            pltpu.sync_copy(dsts.at[pl.ds(ebase + g * _B, _B)], idx_d)
            pltpu.sync_copy(dsts.at[pl.ds(ebase + g * _B, _B)], idx_dx.at[pl.ds(0, _B)])
            cl = pltpu.async_copy(xl.at[idx_s], rows_l, sem_l)
            cr = pltpu.async_copy(xr.at[idx_d], rows_r, sem_r)
            cl.wait()
            cr.wait()
            return pl.loop(0, _B, init_carry=carry)(edge_step)

        init = (jnp.int32(-1), jnp.int32(0), negv, zv)
        fin = pl.loop(0, nb, init_carry=init)(batch_step)
        cur, nfl, m, sv = fin
        lane2 = lax.iota(jnp.int32, 16)

        @pl.when(nfl == 0)
        def _():
            pmeta[...] = jnp.where(lane2 == 0, cur, jnp.full((16,), -1, jnp.int32))
            stash_partial(0, m, sv)

        @pl.when(nfl > 0)
        def _():
            pmeta[...] = jnp.where(lane2 == 1, cur, pmeta[...])
            stash_partial(1, m, sv)

        pltpu.sync_copy(pmeta, pidx_hbm.at[pl.ds(wid * 16, 16)])

    edge_call = pl.kernel(
        edge_body,
        out_type=(
            jax.ShapeDtypeStruct((n * _D,), jnp.float32),
            jax.ShapeDtypeStruct((_NW * 16,), jnp.int32),
            jax.ShapeDtypeStruct((_NW * 2 * 512,), jnp.float32),
        ),
        mesh=mesh,
        scratch_types=[
            pltpu.VMEM((_B,), jnp.int32),
            pltpu.VMEM((_B,), jnp.int32),
            pltpu.VMEM((_B + 16,), jnp.int32),
            pltpu.VMEM((_B, _WPAD), jnp.float32),
            pltpu.VMEM((_B, _WPAD), jnp.float32),
            pltpu.VMEM((_HD,), jnp.float32),
            pltpu.VMEM((_D,), jnp.float32),
            pltpu.VMEM((_D,), jnp.float32),
            pltpu.VMEM((512,), jnp.float32),
            pltpu.VMEM((16,), jnp.int32),
            pltpu.VMEM((28, 16), jnp.float32),
            pltpu.SemaphoreType.DMA,
            pltpu.SemaphoreType.DMA,
        ],
        compiler_params=pltpu.CompilerParams(needs_layout_passes=False),
        name="gatv2_edge_phase",
    )

    words_per_tile = n * _D // _NS
    copy_chunk = words_per_tile // 4

    def fixup_body(out_in, pidx_hbm, pdata_hbm, biasf,
                   out_fin, cbuf, ibuf, pbuf, biasb, ostage, accb):
        c = lax.axis_index("c")
        s_ax = lax.axis_index("s")
        zv = jnp.zeros((16,), jnp.float32)
        negv = jnp.full((16,), _NEG, jnp.float32)

        @pl.when(c == 0)
        def _():
            base = s_ax * words_per_tile
            for ch in range(4):
                r0 = base + ch * copy_chunk
                pltpu.sync_copy(out_in.at[pl.ds(r0, copy_chunk)], cbuf)
                pltpu.sync_copy(cbuf, out_fin.at[pl.ds(r0, copy_chunk)])

        plsc.subcore_barrier()

        @pl.when((c == 0) & (s_ax == 0))
        def _():
            pltpu.sync_copy(biasf, biasb)
            lane = lax.iota(jnp.int32, 16)

            def slot_step(k, carry):
                cur, m, sv = carry
                w = k // 2
                j = k - 2 * w
                pltpu.sync_copy(pidx_hbm.at[pl.ds(w * 16, 16)], ibuf.at[pl.ds(0, 16)])
                d = ibuf[pl.ds(j, 16)][0]
                valid = d >= 0
                pltpu.sync_copy(pdata_hbm.at[pl.ds(k * 512, 512)], pbuf)
                pm = pbuf[pl.ds(0, 16)]
                ps = pbuf[pl.ds(16, 16)]
                same = valid & (d == cur)

                @pl.when(valid & (d != cur) & (cur >= 0) & (cur < n))
                def _():
                    _finalize_row(lane, sv, accb, biasb, ostage, out_fin, cur)

                mprev = jnp.where(same, m, negv)
                sprev = jnp.where(same, sv, zv)
                m2 = jnp.maximum(mprev, pm)
                f1 = jnp.exp(mprev - m2)
                f2 = jnp.exp(pm - m2)
                s2 = sprev * f1 + ps * f2
                for h in range(_H):
                    f1h = _masked_lane_sum(lane, h, f1)
                    f2h = _masked_lane_sum(lane, h, f2)
                    for q in range(4):
                        jj = 4 * h + q
                        ap = jnp.where(same, accb[jj], zv)
                        pa = pbuf[pl.ds(32 + 16 * jj, 16)]
                        nv = ap * f1h + pa * f2h
                        accb[jj] = jnp.where(valid, nv, accb[jj])
                cur2 = jnp.where(valid, d, cur)
                m3 = jnp.where(valid, m2, m)
                s3 = jnp.where(valid, s2, sv)
                return (cur2, m3, s3)

            init = (jnp.int32(-1), negv, zv)
            fin = pl.loop(0, 2 * _NW, init_carry=init)(slot_step)
            cur, m, sv = fin

            @pl.when((cur >= 0) & (cur < n))
            def _():
                _finalize_row(lane, sv, accb, biasb, ostage, out_fin, cur)

    fixup_call = pl.kernel(
        fixup_body,
        out_type=jax.ShapeDtypeStruct((n * _D,), jnp.float32),
        mesh=mesh,
        scratch_types=[
            pltpu.VMEM((copy_chunk,), jnp.float32),
            pltpu.VMEM((32,), jnp.int32),
            pltpu.VMEM((512,), jnp.float32),
            pltpu.VMEM((_D,), jnp.float32),
            pltpu.VMEM((_D,), jnp.float32),
            pltpu.VMEM((28, 16), jnp.float32),
        ],
        compiler_params=pltpu.CompilerParams(needs_layout_passes=False),
        name="gatv2_edge_fixup",
    )

    return edge_call, fixup_call


def kernel(x, edge_index, fc1, convs, fc2):
    n = x.shape[0]
    e_all = edge_index.shape[1] + n
    eb = -(-e_all // (_NW * _B)) * _B
    e_pad = eb * _NW
    nb = eb // _B

    loops = jnp.arange(n, dtype=edge_index.dtype)
    src_all = jnp.concatenate([edge_index[0], loops])
    dst_all = jnp.concatenate([edge_index[1], loops])
    srcp = jnp.pad(src_all, (0, e_pad - e_all))
    dstp = jnp.pad(dst_all, (0, e_pad - e_all), constant_values=n)
    dsts, srcs = lax.sort((dstp, srcp), num_keys=1)

    edge_call, fixup_call = _make_edge_kernels(n, eb, nb)

    h = _fc1_forward(x, fc1)
    for (wl, bl, wr, br, att, bias) in convs:
        xl, xr = _conv_proj(h, wl, bl, wr, br)
        out_part, pidx, pdata = edge_call(xl, xr, srcs, dsts, att.reshape(-1), bias)
        h = fixup_call(out_part, pidx, pdata, bias).reshape(n, _D)
    return _fc2_forward(h, fc2)


# speedup vs baseline: 27.2171x; 1.3177x over previous
import functools

import jax
import jax.numpy as jnp
from jax import lax
from jax.experimental import pallas as pl
from jax.experimental.pallas import tpu as pltpu
from jax.experimental.pallas import tpu_sc as plsc

_H = 7
_D = 64
_HD = _H * _D
_NW = 32
_NS = 16
_B = 56
_NEG = -1e30
_WPAD = 512


def _pad_rows(a, bm):
    m = a.shape[0]
    mp = ((m + bm - 1) // bm) * bm
    if mp == m:
        return a
    return jnp.pad(a, ((0, mp - m), (0, 0)))


def _mlp3_block(x_ref, w1_ref, b1_ref, w2_ref, b2_ref, w3_ref, b3_ref, o_ref):
    h = jnp.maximum(jnp.dot(x_ref[:], w1_ref[:], preferred_element_type=jnp.float32) + b1_ref[:], 0.0)
    h = jnp.maximum(jnp.dot(h, w2_ref[:], preferred_element_type=jnp.float32) + b2_ref[:], 0.0)
    h = jnp.maximum(jnp.dot(h, w3_ref[:], preferred_element_type=jnp.float32) + b3_ref[:], 0.0)
    o_ref[:] = h


def _fc1_forward(x, fc1):
    (w1, b1), (w2, b2), (w3, b3) = fc1
    bm = 512
    xp = _pad_rows(x, bm)
    mp = xp.shape[0]
    full = lambda i: (0, 0)
    return pl.pallas_call(
        _mlp3_block,
        grid=(mp // bm,),
        in_specs=[
            pl.BlockSpec((bm, xp.shape[1]), lambda i: (i, 0)),
            pl.BlockSpec((w1.shape[1], w1.shape[0]), full),
            pl.BlockSpec((1, w1.shape[0]), full),
            pl.BlockSpec((w2.shape[1], w2.shape[0]), full),
            pl.BlockSpec((1, w2.shape[0]), full),
            pl.BlockSpec((w3.shape[1], w3.shape[0]), full),
            pl.BlockSpec((1, w3.shape[0]), full),
        ],
        out_specs=pl.BlockSpec((bm, w3.shape[0]), lambda i: (i, 0)),
        out_shape=jax.ShapeDtypeStruct((mp, w3.shape[0]), jnp.float32),
    )(xp, w1.T, b1[None, :], w2.T, b2[None, :], w3.T, b3[None, :])


def _proj2_block(h_ref, w_ref, b_ref, ol_ref, or_ref):
    o = jnp.dot(h_ref[:], w_ref[:], preferred_element_type=jnp.float32) + b_ref[:]
    z = jnp.zeros((o.shape[0], _WPAD - _HD), jnp.float32)
    ol_ref[:] = jnp.concatenate([o[:, :_HD], z], axis=1)
    or_ref[:] = jnp.concatenate([o[:, _HD:], z], axis=1)


def _conv_proj(h, wl, bl, wr, br):
    wcat = jnp.concatenate([wl, wr], axis=0)
    bcat = jnp.concatenate([bl, br], axis=0)
    bm = 1024
    hp = _pad_rows(h, bm)
    mp = hp.shape[0]
    return pl.pallas_call(
        _proj2_block,
        grid=(mp // bm,),
        in_specs=[
            pl.BlockSpec((bm, hp.shape[1]), lambda i: (i, 0)),
            pl.BlockSpec((hp.shape[1], 2 * _HD), lambda i: (0, 0)),
            pl.BlockSpec((1, 2 * _HD), lambda i: (0, 0)),
        ],
        out_specs=[
            pl.BlockSpec((bm, _WPAD), lambda i: (i, 0)),
            pl.BlockSpec((bm, _WPAD), lambda i: (i, 0)),
        ],
        out_shape=[
            jax.ShapeDtypeStruct((mp, _WPAD), jnp.float32),
            jax.ShapeDtypeStruct((mp, _WPAD), jnp.float32),
        ],
    )(hp, wcat.T, bcat[None, :])


def _fc2_block(h_ref, w1_ref, b1_ref, w2_ref, b2_ref, w3_ref, b3_ref, o_ref):
    h = jnp.maximum(h_ref[:], 0.0)
    h = jnp.dot(h, w1_ref[:], preferred_element_type=jnp.float32) + b1_ref[:]
    h = jnp.maximum(h, 0.0)
    h = jnp.dot(h, w2_ref[:], preferred_element_type=jnp.float32) + b2_ref[:]
    h = jnp.maximum(h, 0.0)
    h = jnp.dot(h, w3_ref[:], preferred_element_type=jnp.float32) + b3_ref[:]
    m = jnp.max(h, axis=1, keepdims=True)
    lse = jnp.log(jnp.sum(jnp.exp(h - m), axis=1, keepdims=True)) + m
    o_ref[:] = h - lse


def _fc2_forward(h, fc2):
    (w1, b1), (w2, b2), (w3, b3) = fc2
    bm = 512
    hp = _pad_rows(h, bm)
    mp = hp.shape[0]
    full = lambda i: (0, 0)
    out = pl.pallas_call(
        _fc2_block,
        grid=(mp // bm,),
        in_specs=[
            pl.BlockSpec((bm, h.shape[1]), lambda i: (i, 0)),
            pl.BlockSpec((w1.shape[1], w1.shape[0]), full),
            pl.BlockSpec((1, w1.shape[0]), full),
            pl.BlockSpec((w2.shape[1], w2.shape[0]), full),
            pl.BlockSpec((1, w2.shape[0]), full),
            pl.BlockSpec((w3.shape[1], w3.shape[0]), full),
            pl.BlockSpec((1, w3.shape[0]), full),
        ],
        out_specs=pl.BlockSpec((bm, w3.shape[0]), lambda i: (i, 0)),
        out_shape=jax.ShapeDtypeStruct((mp, w3.shape[0]), jnp.float32),
    )(hp, w1.T, b1[None, :], w2.T, b2[None, :], w3.T, b3[None, :])
    return out[: h.shape[0]]


def _masked_lane_sum(lane, h, v):
    return jnp.sum(jnp.where(lane == h, v, 0.0))


def _finalize_row(lane, svec, accs, biasb, ostage, out_ref, dst_idx):
    rcp = 1.0 / svec
    for q in range(4):
        o = biasb[pl.ds(16 * q, 16)]
        for h in range(_H):
            rh = _masked_lane_sum(lane, h, rcp)
            o = o + accs[4 * h + q] * (rh * (1.0 / _H))
        ostage[pl.ds(16 * q, 16)] = jnp.maximum(o, 0.0)
    pltpu.sync_copy(ostage, out_ref.at[pl.ds(dst_idx * _D, _D)])


def _make_edge_kernels(n, eb, nb):
    mesh = plsc.VectorSubcoreMesh(core_axis_name="c", subcore_axis_name="s")

    def edge_body(xl, xr, srcs, dsts, attf, biasf,
                  out_hbm, pidx_hbm, pdata_hbm,
                  idx_s0, idx_s1, idx_d0, idx_d1, idx_dx0, idx_dx1,
                  rows_l0, rows_l1, rows_r0, rows_r1, attb, biasb,
                  ostage, pstage, pmeta, accb,
                  sem_l0, sem_l1, sem_r0, sem_r1):
        wid = lax.axis_index("c") * _NS + lax.axis_index("s")
        ebase = wid * eb
        pltpu.sync_copy(attf, attb)
        pltpu.sync_copy(biasf, biasb)
        lane = lax.iota(jnp.int32, 16)
        zv = jnp.zeros((16,), jnp.float32)
        negv = jnp.full((16,), _NEG, jnp.float32)
        pmeta[...] = jnp.full((16,), -1, jnp.int32)

        idx_s = (idx_s0, idx_s1)
        idx_d = (idx_d0, idx_d1)
        idx_dx = (idx_dx0, idx_dx1)
        rows_l = (rows_l0, rows_l1)
        rows_r = (rows_r0, rows_r1)
        sem_l = (sem_l0, sem_l1)
        sem_r = (sem_r0, sem_r1)

        def stash_partial(slot, m, sv):
            pstage[pl.ds(0, 16)] = m
            pstage[pl.ds(16, 16)] = sv
            for j in range(28):
                pstage[pl.ds(32 + 16 * j, 16)] = accb[j]
            pltpu.sync_copy(pstage, pdata_hbm.at[pl.ds((wid * 2 + slot) * 512, 512)])

        def issue(g, par):
            pltpu.sync_copy(srcs.at[pl.ds(ebase + g * _B, _B)], idx_s[par])
            pltpu.sync_copy(dsts.at[pl.ds(ebase + g * _B, _B)], idx_d[par])
            pltpu.sync_copy(dsts.at[pl.ds(ebase + g * _B, _B)],
                            idx_dx[par].at[pl.ds(0, _B)])
            pltpu.async_copy(xl.at[idx_s[par]], rows_l[par], sem_l[par])
            pltpu.async_copy(xr.at[idx_d[par]], rows_r[par], sem_r[par])

        def make_edge_step(par):
            rl, rr, idx = rows_l[par], rows_r[par], idx_dx[par]

            def edge_step(e, ec):
                cur, nfl, m, sv = ec
                d = idx[pl.ds(e, 16)][0]
                same = d == cur
                t = zv
                xlv = []
                for h in range(_H):
                    pacc = None
                    for q in range(4):
                        j = 4 * h + q
                        vl = rl[e, pl.ds(16 * j, 16)]
                        vr = rr[e, pl.ds(16 * j, 16)]
                        xlv.append(vl)
                        v = vl + vr
                        v = jnp.maximum(v, 0.0) + 0.2 * jnp.minimum(v, 0.0)
                        av = attb[pl.ds(16 * j, 16)]
                        pacc = v * av if pacc is None else pacc + v * av
                    th = jnp.sum(pacc)
                    t = jnp.where(lane == h, th, t)
                do_flush = jnp.logical_not(same) & (cur >= 0)

                @pl.when(do_flush & (nfl == 0))
                def _():
                    pmeta[...] = jnp.where(lane == 0, cur, pmeta[...])
                    stash_partial(0, m, sv)

                @pl.when(do_flush & (nfl > 0))
                def _():
                    _finalize_row(lane, sv, accb, biasb, ostage, out_hbm, cur)

                mprev = jnp.where(same, m, negv)
                sprev = jnp.where(same, sv, zv)
                m2 = jnp.maximum(mprev, t)
                f = jnp.exp(mprev - m2)
                ex = jnp.exp(t - m2)
                s2 = sprev * f + ex
                for h in range(_H):
                    fh = _masked_lane_sum(lane, h, f)
                    exh = _masked_lane_sum(lane, h, ex)
                    for q in range(4):
                        j = 4 * h + q
                        ap = jnp.where(same, accb[j], zv)
                        accb[j] = ap * fh + xlv[j] * exh
                nfl2 = jnp.where(same | (cur < 0), nfl, nfl + 1)
                return (d, nfl2, m2, s2)

            return edge_step

        edge_steps = (make_edge_step(0), make_edge_step(1))

        def batch_pair(g, carry):
            for par in (0, 1):
                gg = g + par

                @pl.when(gg + 1 < nb)
                def _():
                    issue(gg + 1, 1 - par)

                pltpu.make_async_copy(xl.at[idx_s[par]], rows_l[par], sem_l[par]).wait()
                pltpu.make_async_copy(xr.at[idx_d[par]], rows_r[par], sem_r[par]).wait()
                carry = pl.loop(0, _B, init_carry=carry)(edge_steps[par])
            return carry

        issue(0, 0)
        init = (jnp.int32(-1), jnp.int32(0), negv, zv)
        fin = pl.loop(0, nb, step=2, init_carry=init)(batch_pair)
        cur, nfl, m, sv = fin
        lane2 = lax.iota(jnp.int32, 16)

        @pl.when(nfl == 0)
        def _():
            pmeta[...] = jnp.where(lane2 == 0, cur, jnp.full((16,), -1, jnp.int32))
            stash_partial(0, m, sv)

        @pl.when(nfl > 0)
        def _():
            pmeta[...] = jnp.where(lane2 == 1, cur, pmeta[...])
            stash_partial(1, m, sv)

        pltpu.sync_copy(pmeta, pidx_hbm.at[pl.ds(wid * 16, 16)])

    edge_call = pl.kernel(
        edge_body,
        out_type=(
            jax.ShapeDtypeStruct((n * _D,), jnp.float32),
            jax.ShapeDtypeStruct((_NW * 16,), jnp.int32),
            jax.ShapeDtypeStruct((_NW * 2 * 512,), jnp.float32),
        ),
        mesh=mesh,
        scratch_types=[
            pltpu.VMEM((_B,), jnp.int32),
            pltpu.VMEM((_B,), jnp.int32),
            pltpu.VMEM((_B,), jnp.int32),
            pltpu.VMEM((_B,), jnp.int32),
            pltpu.VMEM((_B + 16,), jnp.int32),
            pltpu.VMEM((_B + 16,), jnp.int32),
            pltpu.VMEM((_B, _WPAD), jnp.float32),
            pltpu.VMEM((_B, _WPAD), jnp.float32),
            pltpu.VMEM((_B, _WPAD), jnp.float32),
            pltpu.VMEM((_B, _WPAD), jnp.float32),
            pltpu.VMEM((_HD,), jnp.float32),
            pltpu.VMEM((_D,), jnp.float32),
            pltpu.VMEM((_D,), jnp.float32),
            pltpu.VMEM((512,), jnp.float32),
            pltpu.VMEM((16,), jnp.int32),
            pltpu.VMEM((28, 16), jnp.float32),
            pltpu.SemaphoreType.DMA,
            pltpu.SemaphoreType.DMA,
            pltpu.SemaphoreType.DMA,
            pltpu.SemaphoreType.DMA,
        ],
        compiler_params=pltpu.CompilerParams(needs_layout_passes=False),
        name="gatv2_edge_phase",
    )

    words_per_tile = n * _D // _NS
    copy_chunk = words_per_tile // 4

    def fixup_body(out_in, pidx_hbm, pdata_hbm, biasf,
                   out_fin, cbuf, ibuf, pbuf, biasb, ostage, accb):
        c = lax.axis_index("c")
        s_ax = lax.axis_index("s")
        zv = jnp.zeros((16,), jnp.float32)
        negv = jnp.full((16,), _NEG, jnp.float32)

        @pl.when(c == 0)
        def _():
            base = s_ax * words_per_tile
            for ch in range(4):
                r0 = base + ch * copy_chunk
                pltpu.sync_copy(out_in.at[pl.ds(r0, copy_chunk)], cbuf)
                pltpu.sync_copy(cbuf, out_fin.at[pl.ds(r0, copy_chunk)])

        plsc.subcore_barrier()

        @pl.when((c == 0) & (s_ax == 0))
        def _():
            pltpu.sync_copy(biasf, biasb)
            lane = lax.iota(jnp.int32, 16)

            def slot_step(k, carry):
                cur, m, sv = carry
                w = k // 2
                j = k - 2 * w
                pltpu.sync_copy(pidx_hbm.at[pl.ds(w * 16, 16)], ibuf.at[pl.ds(0, 16)])
                d = ibuf[pl.ds(j, 16)][0]
                valid = d >= 0
                pltpu.sync_copy(pdata_hbm.at[pl.ds(k * 512, 512)], pbuf)
                pm = pbuf[pl.ds(0, 16)]
                ps = pbuf[pl.ds(16, 16)]
                same = valid & (d == cur)

                @pl.when(valid & (d != cur) & (cur >= 0) & (cur < n))
                def _():
                    _finalize_row(lane, sv, accb, biasb, ostage, out_fin, cur)

                mprev = jnp.where(same, m, negv)
                sprev = jnp.where(same, sv, zv)
                m2 = jnp.maximum(mprev, pm)
                f1 = jnp.exp(mprev - m2)
                f2 = jnp.exp(pm - m2)
                s2 = sprev * f1 + ps * f2
                for h in range(_H):
                    f1h = _masked_lane_sum(lane, h, f1)
                    f2h = _masked_lane_sum(lane, h, f2)
                    for q in range(4):
                        jj = 4 * h + q
                        ap = jnp.where(same, accb[jj], zv)
                        pa = pbuf[pl.ds(32 + 16 * jj, 16)]
                        nv = ap * f1h + pa * f2h
                        accb[jj] = jnp.where(valid, nv, accb[jj])
                cur2 = jnp.where(valid, d, cur)
                m3 = jnp.where(valid, m2, m)
                s3 = jnp.where(valid, s2, sv)
                return (cur2, m3, s3)

            init = (jnp.int32(-1), negv, zv)
            fin = pl.loop(0, 2 * _NW, init_carry=init)(slot_step)
            cur, m, sv = fin

            @pl.when((cur >= 0) & (cur < n))
            def _():
                _finalize_row(lane, sv, accb, biasb, ostage, out_fin, cur)

    fixup_call = pl.kernel(
        fixup_body,
        out_type=jax.ShapeDtypeStruct((n * _D,), jnp.float32),
        mesh=mesh,
        scratch_types=[
            pltpu.VMEM((copy_chunk,), jnp.float32),
            pltpu.VMEM((32,), jnp.int32),
            pltpu.VMEM((512,), jnp.float32),
            pltpu.VMEM((_D,), jnp.float32),
            pltpu.VMEM((_D,), jnp.float32),
            pltpu.VMEM((28, 16), jnp.float32),
        ],
        compiler_params=pltpu.CompilerParams(needs_layout_passes=False),
        name="gatv2_edge_fixup",
    )

    return edge_call, fixup_call


def kernel(x, edge_index, fc1, convs, fc2):
    n = x.shape[0]
    e_all = edge_index.shape[1] + n
    eb = -(-e_all // (_NW * 2 * _B)) * 2 * _B
    e_pad = eb * _NW
    nb = eb // _B

    loops = jnp.arange(n, dtype=edge_index.dtype)
    src_all = jnp.concatenate([edge_index[0], loops])
    dst_all = jnp.concatenate([edge_index[1], loops])
    srcp = jnp.pad(src_all, (0, e_pad - e_all))
    dstp = jnp.pad(dst_all, (0, e_pad - e_all), constant_values=n)
    dsts, srcs = lax.sort((dstp, srcp), num_keys=1)

    edge_call, fixup_call = _make_edge_kernels(n, eb, nb)

    h = _fc1_forward(x, fc1)
    for (wl, bl, wr, br, att, bias) in convs:
        xl, xr = _conv_proj(h, wl, bl, wr, br)
        out_part, pidx, pdata = edge_call(xl, xr, srcs, dsts, att.reshape(-1), bias)
        h = fixup_call(out_part, pidx, pdata, bias).reshape(n, _D)
    return _fc2_forward(h, fc2)
